# initial kernel scaffold (unmeasured)
import jax
import jax.numpy as jnp
from jax import lax
from jax.experimental import pallas as pl
from jax.experimental.pallas import tpu as pltpu


def kernel(A, B):
    m, k = A.shape
    k2, n = B.shape
    assert k == k2

    def body(a_ref, b_ref, out_ref, send_buf, recv_buf, send_sem, recv_sem):
        my_x = lax.axis_index("x")
        my_y = lax.axis_index("y")
        partner = (1 - my_x, my_y)

        barrier = pltpu.get_barrier_semaphore()
        pl.semaphore_signal(
            barrier, inc=1, device_id=partner,
            device_id_type=pl.DeviceIdType.MESH,
        )
        pl.semaphore_wait(barrier, 1)

        partial = jnp.dot(
            a_ref[...].astype(jnp.bfloat16),
            b_ref[...].astype(jnp.bfloat16),
            preferred_element_type=jnp.float32,
        )
        out_ref[...] = partial
        send_buf[...] = partial.astype(jnp.bfloat16)

        rdma = pltpu.make_async_remote_copy(
            src_ref=send_buf,
            dst_ref=recv_buf,
            send_sem=send_sem,
            recv_sem=recv_sem,
            device_id=partner,
            device_id_type=pl.DeviceIdType.MESH,
        )
        rdma.start()
        rdma.wait()

        out_ref[...] = out_ref[...] + recv_buf[...].astype(jnp.float32)

    return pl.pallas_call(
        body,
        out_shape=jax.ShapeDtypeStruct((m, n), jnp.float32),
        in_specs=[
            pl.BlockSpec(memory_space=pltpu.VMEM),
            pl.BlockSpec(memory_space=pltpu.VMEM),
        ],
        out_specs=pl.BlockSpec(memory_space=pltpu.VMEM),
        scratch_shapes=[
            pltpu.VMEM((m, n), jnp.bfloat16),
            pltpu.VMEM((m, n), jnp.bfloat16),
            pltpu.SemaphoreType.DMA,
            pltpu.SemaphoreType.DMA,
        ],
        compiler_params=pltpu.CompilerParams(collective_id=0),
    )(A, B)


# baseline (device time: 120621 ns/iter reference)
import jax
import jax.numpy as jnp
from jax import lax
from jax.experimental import pallas as pl
from jax.experimental.pallas import tpu as pltpu

N_CHUNKS = 4


def kernel(A, B):
    m, k = A.shape
    k2, n = B.shape
    assert k == k2
    mc = m // N_CHUNKS

    def body(a_ref, b_ref, out_ref, send_buf, recv_buf, send_sems, recv_sems):
        my_x = lax.axis_index("x")
        my_y = lax.axis_index("y")
        partner = (1 - my_x, my_y)

        barrier = pltpu.get_barrier_semaphore()
        pl.semaphore_signal(
            barrier, inc=1, device_id=partner,
            device_id_type=pl.DeviceIdType.MESH,
        )
        pl.semaphore_wait(barrier, 1)

        b_bf16 = b_ref[...].astype(jnp.bfloat16)

        def chunk_rdma(c):
            return pltpu.make_async_remote_copy(
                src_ref=send_buf.at[c],
                dst_ref=recv_buf.at[c],
                send_sem=send_sems.at[c],
                recv_sem=recv_sems.at[c],
                device_id=partner,
                device_id_type=pl.DeviceIdType.MESH,
            )

        for c in range(N_CHUNKS):
            partial = jnp.dot(
                a_ref[pl.ds(c * mc, mc), :].astype(jnp.bfloat16),
                b_bf16,
                preferred_element_type=jnp.float32,
            )
            out_ref[pl.ds(c * mc, mc), :] = partial
            send_buf[c] = partial.astype(jnp.bfloat16)
            chunk_rdma(c).start()

        for c in range(N_CHUNKS):
            rdma = chunk_rdma(c)
            rdma.wait_recv()
            out_ref[pl.ds(c * mc, mc), :] = (
                out_ref[pl.ds(c * mc, mc), :]
                + recv_buf[c].astype(jnp.float32)
            )
            rdma.wait_send()

    return pl.pallas_call(
        body,
        out_shape=jax.ShapeDtypeStruct((m, n), jnp.float32),
        in_specs=[
            pl.BlockSpec(memory_space=pltpu.VMEM),
            pl.BlockSpec(memory_space=pltpu.VMEM),
        ],
        out_specs=pl.BlockSpec(memory_space=pltpu.VMEM),
        scratch_shapes=[
            pltpu.VMEM((N_CHUNKS, mc, n), jnp.bfloat16),
            pltpu.VMEM((N_CHUNKS, mc, n), jnp.bfloat16),
            pltpu.SemaphoreType.DMA((N_CHUNKS,)),
            pltpu.SemaphoreType.DMA((N_CHUNKS,)),
        ],
        compiler_params=pltpu.CompilerParams(
            collective_id=0,
            vmem_limit_bytes=100 * 1024 * 1024,
        ),
    )(A, B)


# device time: 119469 ns/iter; 1.0096x vs baseline; 1.0096x over previous
import jax
import jax.numpy as jnp
from jax import lax
from jax.experimental import pallas as pl
from jax.experimental.pallas import tpu as pltpu

N_CHUNKS = 8


def kernel(A, B):
    m, k = A.shape
    k2, n = B.shape
    assert k == k2
    mc = m // N_CHUNKS

    def body(a_ref, b_ref, out_ref, send_buf, recv_buf, send_sems, recv_sems):
        my_x = lax.axis_index("x")
        my_y = lax.axis_index("y")
        partner = (1 - my_x, my_y)

        barrier = pltpu.get_barrier_semaphore()
        pl.semaphore_signal(
            barrier, inc=1, device_id=partner,
            device_id_type=pl.DeviceIdType.MESH,
        )
        pl.semaphore_wait(barrier, 1)

        b_bf16 = b_ref[...].astype(jnp.bfloat16)

        def chunk_rdma(c):
            return pltpu.make_async_remote_copy(
                src_ref=send_buf.at[c],
                dst_ref=recv_buf.at[c],
                send_sem=send_sems.at[c],
                recv_sem=recv_sems.at[c],
                device_id=partner,
                device_id_type=pl.DeviceIdType.MESH,
            )

        for c in range(N_CHUNKS):
            partial = jnp.dot(
                a_ref[pl.ds(c * mc, mc), :].astype(jnp.bfloat16),
                b_bf16,
                preferred_element_type=jnp.float32,
            )
            out_ref[pl.ds(c * mc, mc), :] = partial
            send_buf[c] = partial.astype(jnp.bfloat16)
            chunk_rdma(c).start()

        for c in range(N_CHUNKS):
            rdma = chunk_rdma(c)
            rdma.wait_recv()
            out_ref[pl.ds(c * mc, mc), :] = (
                out_ref[pl.ds(c * mc, mc), :]
                + recv_buf[c].astype(jnp.float32)
            )
            rdma.wait_send()

    return pl.pallas_call(
        body,
        out_shape=jax.ShapeDtypeStruct((m, n), jnp.float32),
        in_specs=[
            pl.BlockSpec(memory_space=pltpu.VMEM),
            pl.BlockSpec(memory_space=pltpu.VMEM),
        ],
        out_specs=pl.BlockSpec(memory_space=pltpu.VMEM),
        scratch_shapes=[
            pltpu.VMEM((N_CHUNKS, mc, n), jnp.bfloat16),
            pltpu.VMEM((N_CHUNKS, mc, n), jnp.bfloat16),
            pltpu.SemaphoreType.DMA((N_CHUNKS,)),
            pltpu.SemaphoreType.DMA((N_CHUNKS,)),
        ],
        compiler_params=pltpu.CompilerParams(
            collective_id=0,
            vmem_limit_bytes=100 * 1024 * 1024,
        ),
    )(A, B)


# device time: 115022 ns/iter; 1.0487x vs baseline; 1.0387x over previous
import jax
import jax.numpy as jnp
from jax import lax
from jax.experimental import pallas as pl
from jax.experimental.pallas import tpu as pltpu

N_CHUNKS = 8


def kernel(A, B):
    m, k = A.shape
    k2, n = B.shape
    assert k == k2
    mc = m // N_CHUNKS

    def body(a_ref, b_ref, out_ref, send_buf, recv_buf, stage,
             send_sems, recv_sems, copy_sems):
        my_x = lax.axis_index("x")
        my_y = lax.axis_index("y")
        partner = (1 - my_x, my_y)

        barrier = pltpu.get_barrier_semaphore()
        pl.semaphore_signal(
            barrier, inc=1, device_id=partner,
            device_id_type=pl.DeviceIdType.MESH,
        )
        pl.semaphore_wait(barrier, 1)

        b_bf16 = b_ref[...].astype(jnp.bfloat16)

        def chunk_rdma(c):
            return pltpu.make_async_remote_copy(
                src_ref=send_buf.at[c],
                dst_ref=recv_buf.at[c],
                send_sem=send_sems.at[c],
                recv_sem=recv_sems.at[c],
                device_id=partner,
                device_id_type=pl.DeviceIdType.MESH,
            )

        for c in range(N_CHUNKS):
            send_buf[c] = jnp.dot(
                a_ref[pl.ds(c * mc, mc), :].astype(jnp.bfloat16),
                b_bf16,
                preferred_element_type=jnp.float32,
            ).astype(jnp.bfloat16)
            chunk_rdma(c).start()

        copies = [None, None]
        for c in range(N_CHUNKS):
            rdma = chunk_rdma(c)
            rdma.wait_recv()
            slot = c % 2
            if copies[slot] is not None:
                copies[slot].wait()
            stage[slot] = (
                send_buf[c].astype(jnp.float32)
                + recv_buf[c].astype(jnp.float32)
            )
            cp = pltpu.make_async_copy(
                stage.at[slot],
                out_ref.at[pl.ds(c * mc, mc), :],
                copy_sems.at[slot],
            )
            cp.start()
            copies[slot] = cp
            rdma.wait_send()
        for cp in copies:
            cp.wait()

    return pl.pallas_call(
        body,
        out_shape=jax.ShapeDtypeStruct((m, n), jnp.float32),
        in_specs=[
            pl.BlockSpec(memory_space=pltpu.VMEM),
            pl.BlockSpec(memory_space=pltpu.VMEM),
        ],
        out_specs=pl.BlockSpec(memory_space=pltpu.HBM),
        scratch_shapes=[
            pltpu.VMEM((N_CHUNKS, mc, n), jnp.bfloat16),
            pltpu.VMEM((N_CHUNKS, mc, n), jnp.bfloat16),
            pltpu.VMEM((2, mc, n), jnp.float32),
            pltpu.SemaphoreType.DMA((N_CHUNKS,)),
            pltpu.SemaphoreType.DMA((N_CHUNKS,)),
            pltpu.SemaphoreType.DMA((2,)),
        ],
        compiler_params=pltpu.CompilerParams(
            collective_id=0,
            vmem_limit_bytes=100 * 1024 * 1024,
        ),
    )(A, B)


# device time: 114119 ns/iter; 1.0570x vs baseline; 1.0079x over previous
import jax
import jax.numpy as jnp
from jax import lax
from jax.experimental import pallas as pl
from jax.experimental.pallas import tpu as pltpu

N_CHUNKS = 16


def kernel(A, B):
    m, k = A.shape
    k2, n = B.shape
    assert k == k2
    mc = m // N_CHUNKS

    def body(a_ref, b_ref, out_ref, send_buf, recv_buf, stage,
             send_sems, recv_sems, copy_sems):
        my_x = lax.axis_index("x")
        my_y = lax.axis_index("y")
        partner = (1 - my_x, my_y)

        barrier = pltpu.get_barrier_semaphore()
        pl.semaphore_signal(
            barrier, inc=1, device_id=partner,
            device_id_type=pl.DeviceIdType.MESH,
        )
        pl.semaphore_wait(barrier, 1)

        b_bf16 = b_ref[...].astype(jnp.bfloat16)

        def chunk_rdma(c):
            return pltpu.make_async_remote_copy(
                src_ref=send_buf.at[c],
                dst_ref=recv_buf.at[c],
                send_sem=send_sems.at[c],
                recv_sem=recv_sems.at[c],
                device_id=partner,
                device_id_type=pl.DeviceIdType.MESH,
            )

        for c in range(N_CHUNKS):
            send_buf[c] = jnp.dot(
                a_ref[pl.ds(c * mc, mc), :].astype(jnp.bfloat16),
                b_bf16,
                preferred_element_type=jnp.float32,
            ).astype(jnp.bfloat16)
            chunk_rdma(c).start()

        copies = [None, None]
        for c in range(N_CHUNKS):
            rdma = chunk_rdma(c)
            rdma.wait_recv()
            slot = c % 2
            if copies[slot] is not None:
                copies[slot].wait()
            stage[slot] = (
                send_buf[c].astype(jnp.float32)
                + recv_buf[c].astype(jnp.float32)
            )
            cp = pltpu.make_async_copy(
                stage.at[slot],
                out_ref.at[pl.ds(c * mc, mc), :],
                copy_sems.at[slot],
            )
            cp.start()
            copies[slot] = cp
            rdma.wait_send()
        for cp in copies:
            cp.wait()

    return pl.pallas_call(
        body,
        out_shape=jax.ShapeDtypeStruct((m, n), jnp.float32),
        in_specs=[
            pl.BlockSpec(memory_space=pltpu.VMEM),
            pl.BlockSpec(memory_space=pltpu.VMEM),
        ],
        out_specs=pl.BlockSpec(memory_space=pltpu.HBM),
        scratch_shapes=[
            pltpu.VMEM((N_CHUNKS, mc, n), jnp.bfloat16),
            pltpu.VMEM((N_CHUNKS, mc, n), jnp.bfloat16),
            pltpu.VMEM((2, mc, n), jnp.float32),
            pltpu.SemaphoreType.DMA((N_CHUNKS,)),
            pltpu.SemaphoreType.DMA((N_CHUNKS,)),
            pltpu.SemaphoreType.DMA((2,)),
        ],
        compiler_params=pltpu.CompilerParams(
            collective_id=0,
            vmem_limit_bytes=100 * 1024 * 1024,
        ),
    )(A, B)


# device time: 108799 ns/iter; 1.1087x vs baseline; 1.0489x over previous
import jax
import jax.numpy as jnp
from jax import lax
from jax.experimental import pallas as pl
from jax.experimental.pallas import tpu as pltpu

N_CHUNKS = 16


def kernel(A, B):
    m, k = A.shape
    k2, n = B.shape
    assert k == k2
    mc = m // N_CHUNKS

    def body(a_ref, b_ref, out_ref, send_buf, recv_buf, stage,
             send_sems, recv_sems, copy_sems):
        my_x = lax.axis_index("x")
        my_y = lax.axis_index("y")
        partner = (1 - my_x, my_y)

        barrier = pltpu.get_barrier_semaphore()
        pl.semaphore_signal(
            barrier, inc=1, device_id=partner,
            device_id_type=pl.DeviceIdType.MESH,
        )
        pl.semaphore_wait(barrier, 1)

        b_bf16 = b_ref[...].astype(jnp.bfloat16)

        def chunk_rdma(c):
            return pltpu.make_async_remote_copy(
                src_ref=send_buf.at[c],
                dst_ref=recv_buf.at[c],
                send_sem=send_sems.at[c],
                recv_sem=recv_sems.at[c],
                device_id=partner,
                device_id_type=pl.DeviceIdType.MESH,
            )

        for c in range(N_CHUNKS):
            send_buf[c] = jnp.dot(
                a_ref[pl.ds(c * mc, mc), :].astype(jnp.bfloat16),
                b_bf16,
                preferred_element_type=jnp.float32,
            ).astype(jnp.bfloat16)
            chunk_rdma(c).start()

        copies = [None, None]
        for c in range(N_CHUNKS):
            rdma = chunk_rdma(c)
            rdma.wait_recv()
            slot = c % 2
            if copies[slot] is not None:
                copies[slot].wait()
            stage[slot] = (
                send_buf[c].astype(jnp.float32)
                + recv_buf[c].astype(jnp.float32)
            ).astype(jnp.bfloat16)
            cp = pltpu.make_async_copy(
                stage.at[slot],
                out_ref.at[pl.ds(c * mc, mc), :],
                copy_sems.at[slot],
            )
            cp.start()
            copies[slot] = cp
            rdma.wait_send()
        for cp in copies:
            cp.wait()

    return pl.pallas_call(
        body,
        out_shape=jax.ShapeDtypeStruct((m, n), jnp.bfloat16),
        in_specs=[
            pl.BlockSpec(memory_space=pltpu.VMEM),
            pl.BlockSpec(memory_space=pltpu.VMEM),
        ],
        out_specs=pl.BlockSpec(memory_space=pltpu.HBM),
        scratch_shapes=[
            pltpu.VMEM((N_CHUNKS, mc, n), jnp.bfloat16),
            pltpu.VMEM((N_CHUNKS, mc, n), jnp.bfloat16),
            pltpu.VMEM((2, mc, n), jnp.bfloat16),
            pltpu.SemaphoreType.DMA((N_CHUNKS,)),
            pltpu.SemaphoreType.DMA((N_CHUNKS,)),
            pltpu.SemaphoreType.DMA((2,)),
        ],
        compiler_params=pltpu.CompilerParams(
            collective_id=0,
            vmem_limit_bytes=100 * 1024 * 1024,
        ),
    )(A, B)


# device time: 108097 ns/iter; 1.1159x vs baseline; 1.0065x over previous
import jax
import jax.numpy as jnp
from jax import lax
from jax.experimental import pallas as pl
from jax.experimental.pallas import tpu as pltpu

H_CHUNKS = 8


def kernel(A, B):
    m, k = A.shape
    k2, n = B.shape
    assert k == k2
    half = m // 2
    hc = half // H_CHUNKS

    def body(a_ref, b_ref, out_ref, send_raw, recv_raw, my_part, red,
             raw_send_sems, raw_recv_sems, red_send_sems, red_recv_sems,
             copy_sems):
        my_x = lax.axis_index("x")
        my_y = lax.axis_index("y")
        partner = (1 - my_x, my_y)
        m_off = my_x * half
        p_off = (1 - my_x) * half

        barrier = pltpu.get_barrier_semaphore()
        pl.semaphore_signal(
            barrier, inc=1, device_id=partner,
            device_id_type=pl.DeviceIdType.MESH,
        )
        pl.semaphore_wait(barrier, 1)

        b_bf16 = b_ref[...].astype(jnp.bfloat16)

        def raw_rdma(c):
            return pltpu.make_async_remote_copy(
                src_ref=send_raw.at[c],
                dst_ref=recv_raw.at[c],
                send_sem=raw_send_sems.at[c],
                recv_sem=raw_recv_sems.at[c],
                device_id=partner,
                device_id_type=pl.DeviceIdType.MESH,
            )

        def red_out_rdma(c, rows_off):
            return pltpu.make_async_remote_copy(
                src_ref=red.at[c],
                dst_ref=out_ref.at[pl.ds(rows_off + c * hc, hc), :],
                send_sem=red_send_sems.at[c],
                recv_sem=red_recv_sems.at[c],
                device_id=partner,
                device_id_type=pl.DeviceIdType.MESH,
            )

        for c in range(H_CHUNKS):
            send_raw[c] = jnp.dot(
                a_ref[pl.ds(p_off + c * hc, hc), :].astype(jnp.bfloat16),
                b_bf16,
                preferred_element_type=jnp.float32,
            ).astype(jnp.bfloat16)
            raw_rdma(c).start()

        for c in range(H_CHUNKS):
            my_part[c] = jnp.dot(
                a_ref[pl.ds(m_off + c * hc, hc), :].astype(jnp.bfloat16),
                b_bf16,
                preferred_element_type=jnp.float32,
            ).astype(jnp.bfloat16)

        for c in range(H_CHUNKS):
            rdma = raw_rdma(c)
            rdma.wait_recv()
            red[c] = (
                my_part[c].astype(jnp.float32)
                + recv_raw[c].astype(jnp.float32)
            ).astype(jnp.bfloat16)
            pltpu.make_async_copy(
                red.at[c],
                out_ref.at[pl.ds(m_off + c * hc, hc), :],
                copy_sems.at[c],
            ).start()
            red_out_rdma(c, m_off).start()
            rdma.wait_send()

        for c in range(H_CHUNKS):
            inbound = red_out_rdma(c, p_off)
            inbound.wait_recv()
            inbound.wait_send()
            pltpu.make_async_copy(
                red.at[c],
                out_ref.at[pl.ds(m_off + c * hc, hc), :],
                copy_sems.at[c],
            ).wait()

    return pl.pallas_call(
        body,
        out_shape=jax.ShapeDtypeStruct((m, n), jnp.bfloat16),
        in_specs=[
            pl.BlockSpec(memory_space=pltpu.VMEM),
            pl.BlockSpec(memory_space=pltpu.VMEM),
        ],
        out_specs=pl.BlockSpec(memory_space=pltpu.HBM),
        scratch_shapes=[
            pltpu.VMEM((H_CHUNKS, hc, n), jnp.bfloat16),
            pltpu.VMEM((H_CHUNKS, hc, n), jnp.bfloat16),
            pltpu.VMEM((H_CHUNKS, hc, n), jnp.bfloat16),
            pltpu.VMEM((H_CHUNKS, hc, n), jnp.bfloat16),
            pltpu.SemaphoreType.DMA((H_CHUNKS,)),
            pltpu.SemaphoreType.DMA((H_CHUNKS,)),
            pltpu.SemaphoreType.DMA((H_CHUNKS,)),
            pltpu.SemaphoreType.DMA((H_CHUNKS,)),
            pltpu.SemaphoreType.DMA((H_CHUNKS,)),
        ],
        compiler_params=pltpu.CompilerParams(
            collective_id=0,
            vmem_limit_bytes=100 * 1024 * 1024,
        ),
    )(A, B)
